# baseline (speedup 1.0000x reference)
import functools

import jax
import jax.numpy as jnp
from jax import lax
from jax.experimental import pallas as pl
from jax.experimental.pallas import tpu as pltpu
from jax.experimental.pallas import tpu_sc as plsc

_NPAD = 64
_TPAD = 128
_RB = 5000


def _lse_body(x_ref, out_ref, m_ref, s_ref):
    j = pl.program_id(0)
    nj = pl.num_programs(0)
    xb = x_ref[...]
    bm = jnp.max(xb, axis=0, keepdims=True)
    bs = jnp.sum(jnp.exp(xb - bm), axis=0, keepdims=True)
    m_ref[pl.ds(j, 1), :] = bm
    s_ref[pl.ds(j, 1), :] = bs

    @pl.when(j == nj - 1)
    def _():
        ma = m_ref[...]
        sa = s_ref[...]
        m = jnp.max(ma, axis=0, keepdims=True)
        s = jnp.sum(sa * jnp.exp(ma - m), axis=0, keepdims=True)
        out_ref[...] = m + jnp.log(s)


def _row_lse(xt):
    c, b = xt.shape
    assert c % _RB == 0
    nj = c // _RB
    return pl.pallas_call(
        _lse_body,
        grid=(nj,),
        in_specs=[pl.BlockSpec((_RB, b), lambda j: (j, 0))],
        out_specs=pl.BlockSpec((1, b), lambda j: (0, 0)),
        out_shape=jax.ShapeDtypeStruct((1, b), jnp.float32),
        scratch_shapes=[
            pltpu.VMEM((nj, b), jnp.float32),
            pltpu.VMEM((nj, b), jnp.float32),
        ],
    )(xt)


def _combine_body(lse_ref, vy_ref, pos_ref, vn_ref, out_ref, *, nbatch, k):
    lse = lse_ref[...]
    vy = vy_ref[...]
    pos = pos_ref[...]
    vn = vn_ref[...]
    anchor = pos >= 0
    kmask = lax.broadcasted_iota(jnp.int32, vn.shape, 1) < k
    vnm = jnp.where(kmask & anchor, vn, -jnp.inf)
    m = jnp.maximum(vy, jnp.max(vnm, axis=1, keepdims=True))
    s = jnp.exp(vy - m) + jnp.sum(jnp.exp(vnm - m), axis=1, keepdims=True)
    loss = lse - m - jnp.log(s)
    out_ref[...] = jnp.sum(loss, axis=0, keepdims=True) / nbatch


def _combine(lse, vy, pos, vn, k):
    b = vn.shape[0]
    return pl.pallas_call(
        functools.partial(_combine_body, nbatch=b, k=k),
        out_shape=jax.ShapeDtypeStruct((1, 1), jnp.float32),
    )(lse.reshape(b, 1), vy.reshape(b, 1), pos.reshape(b, 1), vn)


def _make_sc_gather(b, c, npad):
    mesh = plsc.VectorSubcoreMesh(core_axis_name="c", subcore_axis_name="s")
    nc, ns = mesh.num_cores, mesh.num_subcores
    nw = nc * ns
    rp = b // nw
    nch = rp * npad // 128

    @functools.partial(
        pl.kernel,
        out_type=(
            jax.ShapeDtypeStruct((b,), jnp.float32),
            jax.ShapeDtypeStruct((b,), jnp.int32),
            jax.ShapeDtypeStruct((nw, nch, 128), jnp.float32),
        ),
        mesh=mesh,
        scratch_types=(
            pltpu.VMEM((rp,), jnp.int32),
            pltpu.VMEM((rp,), jnp.int32),
            pltpu.VMEM((rp,), jnp.int32),
            pltpu.VMEM((rp, _TPAD), jnp.int32),
            pltpu.VMEM((rp, 128), jnp.float32),
            pltpu.VMEM((npad, 128), jnp.float32),
            pltpu.VMEM((npad, 128), jnp.float32),
            pltpu.VMEM((npad, 128), jnp.float32),
            pltpu.VMEM((npad, 128), jnp.float32),
            pltpu.VMEM((rp,), jnp.float32),
            pltpu.VMEM((nch, 128), jnp.float32),
            pltpu.SemaphoreType.DMA,
            pltpu.SemaphoreType.DMA,
            pltpu.SemaphoreType.DMA,
            pltpu.SemaphoreType.DMA,
            pltpu.SemaphoreType.DMA,
        ),
    )
    def sc_gather(xt, y, apos, aneigh, vy_out, pos_out, vn_out,
                  y_v, pos_v, spos_v, neigh_v, vyd_v, dst0, dst1, dst2,
                  dst3, vy_v, vn_v, sem, sem0, sem1, sem2, sem3):
        wid = lax.axis_index("s") * nc + lax.axis_index("c")
        base = wid * rp
        tile0 = (base // 128) * 128
        smod0 = base - tile0
        io16 = lax.iota(jnp.int32, 16)
        dnums = lax.GatherDimensionNumbers(
            offset_dims=(), collapsed_slice_dims=(0,), start_index_map=(0,))

        def take_splat(seg, lane):
            return lax.gather(
                seg, jnp.full((16, 1), lane, jnp.int32),
                dimension_numbers=dnums, slice_sizes=(1,),
                mode=lax.GatherScatterMode.PROMISE_IN_BOUNDS)

        pltpu.sync_copy(y.at[pl.ds(base, rp)], y_v)
        cp_y = pltpu.async_copy(
            xt.at[y_v, pl.ds(tile0, 128)], vyd_v, sem0)
        pltpu.async_copy(apos.at[y_v], pos_v, sem).wait()
        for ch in range(rp // 16):
            sl = pl.ds(ch * 16, 16)
            spos_v[sl] = jnp.maximum(pos_v[sl], 0)
        pltpu.async_copy(aneigh.at[spos_v], neigh_v, sem).wait()
        dsts = (dst0, dst1, dst2, dst3)
        sems = (sem0, sem1, sem2, sem3)
        nbuf = len(dsts)
        cp_y.wait()
        for ch in range(rp // 16):
            def body_y(jj, acc, ch=ch):
                c0 = smod0 + ch * 16 + jj
                cc = (c0 // 16) * 16
                seg = vyd_v[ch * 16 + jj, pl.ds(cc, 16)]
                return jnp.where(io16 == jj, take_splat(seg, c0 - cc), acc)
            vy_v[pl.ds(ch * 16, 16)] = lax.fori_loop(
                0, 16, body_y, jnp.zeros((16,), jnp.float32))
        cps = [
            pltpu.async_copy(
                xt.at[neigh_v.at[r, pl.ds(0, npad)], pl.ds(tile0, 128)],
                dsts[r % nbuf], sems[r % nbuf])
            for r in range(nbuf)
        ]
        for r in range(rp):
            cps[r % nbuf].wait()
            dst = dsts[r % nbuf]
            c0 = smod0 + r
            cc = (c0 // 16) * 16
            lane = c0 - cc
            for q in range(npad // 16):
                def body_n(jj, acc, q=q, dst=dst):
                    seg = dst[q * 16 + jj, pl.ds(cc, 16)]
                    return jnp.where(io16 == jj, take_splat(seg, lane), acc)
                e = r * npad + q * 16
                vn_v[e // 128, pl.ds(e % 128, 16)] = lax.fori_loop(
                    0, 16, body_n, jnp.zeros((16,), jnp.float32))
            if r + nbuf < rp:
                cps[r % nbuf] = pltpu.async_copy(
                    xt.at[neigh_v.at[r + nbuf, pl.ds(0, npad)],
                          pl.ds(tile0, 128)],
                    dsts[r % nbuf], sems[r % nbuf])
        pltpu.sync_copy(vy_v, vy_out.at[pl.ds(base, rp)])
        pltpu.sync_copy(pos_v, pos_out.at[pl.ds(base, rp)])
        pltpu.sync_copy(vn_v, vn_out.at[wid])

    return sc_gather


def kernel(x, y, ans_position, ans_neighbours):
    b, c = x.shape
    a, k = ans_neighbours.shape
    neigh_pad = jnp.pad(ans_neighbours, ((0, 0), (0, _TPAD - k)))
    xt = x.T
    sc_gather = _make_sc_gather(b, c, _NPAD)
    vy, pos, vn = sc_gather(xt, y, ans_position, neigh_pad)
    lse = _row_lse(xt)
    out = _combine(lse, vy, pos, vn.reshape(b, _NPAD), k)
    return out[0, 0]


# speedup vs baseline: 1.0046x; 1.0046x over previous
import functools

import jax
import jax.numpy as jnp
from jax import lax
from jax.experimental import pallas as pl
from jax.experimental.pallas import tpu as pltpu
from jax.experimental.pallas import tpu_sc as plsc

_NPAD = 64
_TPAD = 128
_RB = 4000


_NSPLIT = 4


def _lse_body(x_hbm, out_ref, buf, sems, m_ref, s_ref):
    j = pl.program_id(0)
    nj = pl.num_programs(0)
    b = out_ref.shape[1]
    rows_per = _RB // _NSPLIT

    def issue(step, slot):
        for i in range(_NSPLIT):
            pltpu.make_async_copy(
                x_hbm.at[pl.ds(step * _RB + i * rows_per, rows_per), :],
                buf.at[slot, pl.ds(i * rows_per, rows_per), :],
                sems.at[slot, i],
            ).start()

    def drain(slot):
        for i in range(_NSPLIT):
            pltpu.make_async_copy(
                x_hbm.at[pl.ds(i * rows_per, rows_per), :],
                buf.at[slot, pl.ds(i * rows_per, rows_per), :],
                sems.at[slot, i],
            ).wait()

    @pl.when(j == 0)
    def _():
        issue(0, 0)

    @pl.when(j + 1 < nj)
    def _():
        issue(j + 1, (j + 1) % 2)

    slot = j % 2
    drain(slot)
    xb = buf[slot]
    bm = jnp.max(xb, axis=0, keepdims=True)
    bs = jnp.sum(jnp.exp(xb - bm), axis=0, keepdims=True)
    m_ref[pl.ds(j, 1), :] = bm
    s_ref[pl.ds(j, 1), :] = bs

    @pl.when(j == nj - 1)
    def _():
        ma = m_ref[...]
        sa = s_ref[...]
        m = jnp.max(ma, axis=0, keepdims=True)
        s = jnp.sum(sa * jnp.exp(ma - m), axis=0, keepdims=True)
        out_ref[...] = m + jnp.log(s)


def _row_lse(xt):
    c, b = xt.shape
    assert c % _RB == 0
    nj = c // _RB
    return pl.pallas_call(
        _lse_body,
        grid=(nj,),
        in_specs=[pl.BlockSpec(memory_space=pl.ANY)],
        out_specs=pl.BlockSpec((1, b), lambda j: (0, 0)),
        out_shape=jax.ShapeDtypeStruct((1, b), jnp.float32),
        scratch_shapes=[
            pltpu.VMEM((2, _RB, b), jnp.float32),
            pltpu.SemaphoreType.DMA((2, _NSPLIT)),
            pltpu.VMEM((nj, b), jnp.float32),
            pltpu.VMEM((nj, b), jnp.float32),
        ],
    )(xt)


def _combine_body(lse_ref, vy_ref, pos_ref, vn_ref, out_ref, *, nbatch, k):
    lse = lse_ref[...]
    vy = vy_ref[...]
    pos = pos_ref[...]
    vn = vn_ref[...]
    anchor = pos >= 0
    kmask = lax.broadcasted_iota(jnp.int32, vn.shape, 1) < k
    vnm = jnp.where(kmask & anchor, vn, -jnp.inf)
    m = jnp.maximum(vy, jnp.max(vnm, axis=1, keepdims=True))
    s = jnp.exp(vy - m) + jnp.sum(jnp.exp(vnm - m), axis=1, keepdims=True)
    loss = lse - m - jnp.log(s)
    out_ref[...] = jnp.sum(loss, axis=0, keepdims=True) / nbatch


def _combine(lse, vy, pos, vn, k):
    b = vn.shape[0]
    return pl.pallas_call(
        functools.partial(_combine_body, nbatch=b, k=k),
        out_shape=jax.ShapeDtypeStruct((1, 1), jnp.float32),
    )(lse.reshape(b, 1), vy.reshape(b, 1), pos.reshape(b, 1), vn)


def _make_sc_gather(b, c, npad):
    mesh = plsc.VectorSubcoreMesh(core_axis_name="c", subcore_axis_name="s")
    nc, ns = mesh.num_cores, mesh.num_subcores
    nw = nc * ns
    rp = b // nw
    nch = rp * npad // 128

    @functools.partial(
        pl.kernel,
        out_type=(
            jax.ShapeDtypeStruct((b,), jnp.float32),
            jax.ShapeDtypeStruct((b,), jnp.int32),
            jax.ShapeDtypeStruct((nw, nch, 128), jnp.float32),
        ),
        mesh=mesh,
        scratch_types=(
            pltpu.VMEM((rp,), jnp.int32),
            pltpu.VMEM((rp,), jnp.int32),
            pltpu.VMEM((rp,), jnp.int32),
            pltpu.VMEM((rp, _TPAD), jnp.int32),
            pltpu.VMEM((rp, 128), jnp.float32),
            pltpu.VMEM((npad, 128), jnp.float32),
            pltpu.VMEM((npad, 128), jnp.float32),
            pltpu.VMEM((npad, 128), jnp.float32),
            pltpu.VMEM((npad, 128), jnp.float32),
            pltpu.VMEM((rp,), jnp.float32),
            pltpu.VMEM((nch, 128), jnp.float32),
            pltpu.SemaphoreType.DMA,
            pltpu.SemaphoreType.DMA,
            pltpu.SemaphoreType.DMA,
            pltpu.SemaphoreType.DMA,
            pltpu.SemaphoreType.DMA,
        ),
    )
    def sc_gather(xt, y, apos, aneigh, vy_out, pos_out, vn_out,
                  y_v, pos_v, spos_v, neigh_v, vyd_v, dst0, dst1, dst2,
                  dst3, vy_v, vn_v, sem, sem0, sem1, sem2, sem3):
        wid = lax.axis_index("s") * nc + lax.axis_index("c")
        base = wid * rp
        tile0 = (base // 128) * 128
        smod0 = base - tile0
        io16 = lax.iota(jnp.int32, 16)
        dnums = lax.GatherDimensionNumbers(
            offset_dims=(), collapsed_slice_dims=(0,), start_index_map=(0,))

        def take_splat(seg, lane):
            return lax.gather(
                seg, jnp.full((16, 1), lane, jnp.int32),
                dimension_numbers=dnums, slice_sizes=(1,),
                mode=lax.GatherScatterMode.PROMISE_IN_BOUNDS)

        pltpu.sync_copy(y.at[pl.ds(base, rp)], y_v)
        cp_y = pltpu.async_copy(
            xt.at[y_v, pl.ds(tile0, 128)], vyd_v, sem0)
        pltpu.async_copy(apos.at[y_v], pos_v, sem).wait()
        for ch in range(rp // 16):
            sl = pl.ds(ch * 16, 16)
            spos_v[sl] = jnp.maximum(pos_v[sl], 0)
        pltpu.async_copy(aneigh.at[spos_v], neigh_v, sem).wait()
        dsts = (dst0, dst1, dst2, dst3)
        sems = (sem0, sem1, sem2, sem3)
        nbuf = len(dsts)
        cp_y.wait()
        for ch in range(rp // 16):
            def body_y(jj, acc, ch=ch):
                c0 = smod0 + ch * 16 + jj
                cc = (c0 // 16) * 16
                seg = vyd_v[ch * 16 + jj, pl.ds(cc, 16)]
                return jnp.where(io16 == jj, take_splat(seg, c0 - cc), acc)
            vy_v[pl.ds(ch * 16, 16)] = lax.fori_loop(
                0, 16, body_y, jnp.zeros((16,), jnp.float32))
        cps = [
            pltpu.async_copy(
                xt.at[neigh_v.at[r, pl.ds(0, npad)], pl.ds(tile0, 128)],
                dsts[r % nbuf], sems[r % nbuf])
            for r in range(nbuf)
        ]
        for r in range(rp):
            cps[r % nbuf].wait()
            dst = dsts[r % nbuf]
            c0 = smod0 + r
            cc = (c0 // 16) * 16
            lane = c0 - cc
            for q in range(npad // 16):
                def body_n(jj, acc, q=q, dst=dst):
                    seg = dst[q * 16 + jj, pl.ds(cc, 16)]
                    return jnp.where(io16 == jj, take_splat(seg, lane), acc)
                e = r * npad + q * 16
                vn_v[e // 128, pl.ds(e % 128, 16)] = lax.fori_loop(
                    0, 16, body_n, jnp.zeros((16,), jnp.float32))
            if r + nbuf < rp:
                cps[r % nbuf] = pltpu.async_copy(
                    xt.at[neigh_v.at[r + nbuf, pl.ds(0, npad)],
                          pl.ds(tile0, 128)],
                    dsts[r % nbuf], sems[r % nbuf])
        pltpu.sync_copy(vy_v, vy_out.at[pl.ds(base, rp)])
        pltpu.sync_copy(pos_v, pos_out.at[pl.ds(base, rp)])
        pltpu.sync_copy(vn_v, vn_out.at[wid])

    return sc_gather


def kernel(x, y, ans_position, ans_neighbours):
    b, c = x.shape
    a, k = ans_neighbours.shape
    neigh_pad = jnp.pad(ans_neighbours, ((0, 0), (0, _TPAD - k)))
    xt = x.T
    sc_gather = _make_sc_gather(b, c, _NPAD)
    vy, pos, vn = sc_gather(xt, y, ans_position, neigh_pad)
    lse = _row_lse(xt)
    out = _combine(lse, vy, pos, vn.reshape(b, _NPAD), k)
    return out[0, 0]
